# baseline (speedup 1.0000x reference)
import jax
import jax.numpy as jnp
from jax import lax
from jax.experimental import pallas as pl
from jax.experimental.pallas import tpu as pltpu
from jax.experimental.pallas import tpu_sc as plsc

N = 8192
E = 262144
NC = 2
NS = 16
NW = NC * NS
C = E // NW
VPT = C // 16
RB = N // 16
SLICE = N // NW
SR = RB // NW
PWPAD = 272

_mesh = plsc.VectorSubcoreMesh(
    core_axis_name="c", subcore_axis_name="s", num_cores=NC, num_subcores=NS)
_cp = pltpu.CompilerParams(needs_layout_passes=False)


def _spmv_body(key_h, pidx_h, x_h, pw_h, zero_h, part_h,
               key_v, pidx_v, x_v, pw_v, acc_v, sem):
    wid = lax.axis_index("s") * NC + lax.axis_index("c")
    base = wid * C
    cps = [
        pltpu.async_copy(key_h.at[pl.ds(base, C)], key_v, sem),
        pltpu.async_copy(pidx_h.at[pl.ds(base, C)], pidx_v, sem),
        pltpu.async_copy(x_h, x_v, sem),
        pltpu.async_copy(pw_h, pw_v, sem),
        pltpu.async_copy(zero_h, acc_v, sem),
    ]
    for cp in cps:
        cp.wait()

    @plsc.parallel_loop(0, VPT, unroll=16)
    def step(t):
        off = t * 16
        k = key_v[pl.ds(off, 16)]
        pf = pidx_v[pl.ds(off, 16)]
        p = pf.astype(jnp.int32)
        col = lax.bitwise_and(k, N - 1)
        row = lax.shift_right_logical(k, 13)
        rhi = lax.shift_right_logical(row, 4)
        rlo = lax.bitwise_and(row, 15)
        w = plsc.load_gather(pw_v, [p])
        xv = plsc.load_gather(x_v, [col])
        plsc.addupdate_scatter(acc_v, [rhi, rlo], w * xv)

    pltpu.sync_copy(acc_v, part_h.at[pl.ds(wid * RB, RB)])


_spmv_call = pl.kernel(
    _spmv_body,
    out_type=jax.ShapeDtypeStruct((NW * RB, 16), jnp.float32),
    mesh=_mesh,
    scratch_types=[
        pltpu.VMEM((C,), jnp.int32),
        pltpu.VMEM((C,), jnp.float32),
        pltpu.VMEM((N,), jnp.float32),
        pltpu.VMEM((PWPAD,), jnp.float32),
        pltpu.VMEM((RB, 16), jnp.float32),
        pltpu.SemaphoreType.DMA,
    ],
    compiler_params=_cp,
)


def _reduce_body(part_h, pb_h, bidx_h, out_h, stage_v, red_v, bidx_v, pb_v,
                 sem):
    wid = lax.axis_index("s") * NC + lax.axis_index("c")
    rbase = wid * SR
    cps = [
        pltpu.async_copy(part_h.at[pl.ds(s * RB + rbase, SR)],
                         stage_v.at[pl.ds(s * SR, SR)], sem)
        for s in range(NW)
    ]
    cps.append(pltpu.async_copy(bidx_h.at[pl.ds(wid * SLICE, SLICE)],
                                bidx_v, sem))
    cps.append(pltpu.async_copy(pb_h, pb_v, sem))
    for cp in cps:
        cp.wait()

    def fin_step(t, carry):
        v = stage_v[t]
        for s in range(1, NW):
            v = v + stage_v[s * SR + t]
        bi = bidx_v[pl.ds(t * 16, 16)]
        v = v + plsc.load_gather(pb_v, [bi])
        red_v[t] = v
        return carry

    lax.fori_loop(0, SR, fin_step, 0)
    pltpu.sync_copy(red_v, out_h.at[pl.ds(rbase, SR)])


_reduce_call = pl.kernel(
    _reduce_body,
    out_type=jax.ShapeDtypeStruct((RB, 16), jnp.float32),
    mesh=_mesh,
    scratch_types=[
        pltpu.VMEM((NW * SR, 16), jnp.float32),
        pltpu.VMEM((SR, 16), jnp.float32),
        pltpu.VMEM((SLICE,), jnp.int32),
        pltpu.VMEM((16,), jnp.float32),
        pltpu.SemaphoreType.DMA,
    ],
    compiler_params=_cp,
)


def kernel(x, param_w, param_b, weight_ij, weight_pidx, bias_idx):
    xf = x.reshape(-1)
    key = weight_ij[:, 0] * N + weight_ij[:, 1]
    pidx_f = weight_pidx.astype(jnp.float32)
    key_s, pidx_fs = lax.sort((key, pidx_f), dimension=0, num_keys=1,
                              is_stable=False)
    nxt = jnp.concatenate([key_s[1:], jnp.full((1,), -1, key_s.dtype)])
    pidx_eff = jnp.where(key_s != nxt, pidx_fs, jnp.float32(PWPAD - 16))
    pw_pad = jnp.concatenate(
        [param_w, jnp.zeros((PWPAD - param_w.shape[0],), param_w.dtype)])
    pb_pad = jnp.concatenate(
        [param_b, jnp.zeros((16 - param_b.shape[0],), param_b.dtype)])
    zero = jnp.zeros((RB, 16), jnp.float32)
    parts = _spmv_call(key_s, pidx_eff, xf, pw_pad, zero)
    y2d = _reduce_call(parts, pb_pad, bias_idx)
    return y2d.reshape(N)


# speedup vs baseline: 1.0018x; 1.0018x over previous
import jax
import jax.numpy as jnp
from jax import lax
from jax.experimental import pallas as pl
from jax.experimental.pallas import tpu as pltpu
from jax.experimental.pallas import tpu_sc as plsc

N = 8192
E = 262144
NC = 2
NS = 16
NW = NC * NS
C = E // NW
VPT = C // 16
RB = N // 16
SLICE = N // NW
SR = RB // NW
PWPAD = 272

_mesh = plsc.VectorSubcoreMesh(
    core_axis_name="c", subcore_axis_name="s", num_cores=NC, num_subcores=NS)
_cp = pltpu.CompilerParams(needs_layout_passes=False)


def _spmv_body(key_h, pidx_h, x_h, pw_h, zero_h, part_h,
               key_v, pidx_v, x_v, pw_v, acc_v, sem):
    wid = lax.axis_index("s") * NC + lax.axis_index("c")
    base = wid * C
    cps = [
        pltpu.async_copy(key_h.at[pl.ds(base, C)], key_v, sem),
        pltpu.async_copy(pidx_h.at[pl.ds(base, C)], pidx_v, sem),
        pltpu.async_copy(x_h, x_v, sem),
        pltpu.async_copy(pw_h, pw_v, sem),
        pltpu.async_copy(zero_h, acc_v, sem),
    ]
    for cp in cps:
        cp.wait()

    @plsc.parallel_loop(0, VPT, unroll=8)
    def step(t):
        off = t * 16
        k = key_v[pl.ds(off, 16)]
        pf = pidx_v[pl.ds(off, 16)]
        p = pf.astype(jnp.int32)
        col = lax.bitwise_and(k, N - 1)
        row = lax.shift_right_logical(k, 13)
        rhi = lax.shift_right_logical(row, 4)
        rlo = lax.bitwise_and(row, 15)
        w = plsc.load_gather(pw_v, [p])
        xv = plsc.load_gather(x_v, [col])
        plsc.addupdate_scatter(acc_v, [rhi, rlo], w * xv)

    pltpu.sync_copy(acc_v, part_h.at[pl.ds(wid * RB, RB)])


_spmv_call = pl.kernel(
    _spmv_body,
    out_type=jax.ShapeDtypeStruct((NW * RB, 16), jnp.float32),
    mesh=_mesh,
    scratch_types=[
        pltpu.VMEM((C,), jnp.int32),
        pltpu.VMEM((C,), jnp.float32),
        pltpu.VMEM((N,), jnp.float32),
        pltpu.VMEM((PWPAD,), jnp.float32),
        pltpu.VMEM((RB, 16), jnp.float32),
        pltpu.SemaphoreType.DMA,
    ],
    compiler_params=_cp,
)


def _reduce_body(part_h, pb_h, bidx_h, out_h, stage_v, red_v, bidx_v, pb_v,
                 sem):
    wid = lax.axis_index("s") * NC + lax.axis_index("c")
    rbase = wid * SR
    cps = [
        pltpu.async_copy(part_h.at[pl.ds(s * RB + rbase, SR)],
                         stage_v.at[pl.ds(s * SR, SR)], sem)
        for s in range(NW)
    ]
    cps.append(pltpu.async_copy(bidx_h.at[pl.ds(wid * SLICE, SLICE)],
                                bidx_v, sem))
    cps.append(pltpu.async_copy(pb_h, pb_v, sem))
    for cp in cps:
        cp.wait()

    def fin_step(t, carry):
        v = stage_v[t]
        for s in range(1, NW):
            v = v + stage_v[s * SR + t]
        bi = bidx_v[pl.ds(t * 16, 16)]
        v = v + plsc.load_gather(pb_v, [bi])
        red_v[t] = v
        return carry

    lax.fori_loop(0, SR, fin_step, 0)
    pltpu.sync_copy(red_v, out_h.at[pl.ds(rbase, SR)])


_reduce_call = pl.kernel(
    _reduce_body,
    out_type=jax.ShapeDtypeStruct((RB, 16), jnp.float32),
    mesh=_mesh,
    scratch_types=[
        pltpu.VMEM((NW * SR, 16), jnp.float32),
        pltpu.VMEM((SR, 16), jnp.float32),
        pltpu.VMEM((SLICE,), jnp.int32),
        pltpu.VMEM((16,), jnp.float32),
        pltpu.SemaphoreType.DMA,
    ],
    compiler_params=_cp,
)


def kernel(x, param_w, param_b, weight_ij, weight_pidx, bias_idx):
    xf = x.reshape(-1)
    key = weight_ij[:, 0] * N + weight_ij[:, 1]
    pidx_f = weight_pidx.astype(jnp.float32)
    key_s, pidx_fs = lax.sort((key, pidx_f), dimension=0, num_keys=1,
                              is_stable=False)
    nxt = jnp.concatenate([key_s[1:], jnp.full((1,), -1, key_s.dtype)])
    pidx_eff = jnp.where(key_s != nxt, pidx_fs, jnp.float32(PWPAD - 16))
    pw_pad = jnp.concatenate(
        [param_w, jnp.zeros((PWPAD - param_w.shape[0],), param_w.dtype)])
    pb_pad = jnp.concatenate(
        [param_b, jnp.zeros((16 - param_b.shape[0],), param_b.dtype)])
    zero = jnp.zeros((RB, 16), jnp.float32)
    parts = _spmv_call(key_s, pidx_eff, xf, pw_pad, zero)
    y2d = _reduce_call(parts, pb_pad, bias_idx)
    return y2d.reshape(N)
